# baseline (speedup 1.0000x reference)
import functools

import jax
import jax.numpy as jnp
from jax import lax
from jax.experimental import pallas as pl
from jax.experimental.pallas import tpu as pltpu
from jax.experimental.pallas import tpu_sc as plsc

VOCAB = 100000
HID = 128
MAXPOS = 512
B = 1024
L = 200
PAD = 1
EPS = 1e-5

NW = 32
ROWS_PER_W = B // NW
TOK_W = ROWS_PER_W * L
LP = 208
NV = LP // 16
PT = 224
CH = 80
NCH = TOK_W // CH
NG = CH // 16
LANES = 16
HV = HID // LANES
IDS_PAD = TOK_W + LANES

_GATHER_DN = lax.GatherDimensionNumbers(
    offset_dims=(), collapsed_slice_dims=(0,), start_index_map=(0,))


def _take(x, idx):
    return lax.gather(x, idx[:, None], _GATHER_DN, slice_sizes=(1,),
                      mode=lax.GatherScatterMode.PROMISE_IN_BOUNDS)


def _allsum(x):
    iota = lax.iota(jnp.int32, LANES)
    for k in (1, 2, 4, 8):
        x = x + _take(x, iota ^ k)
    return x


def _cumsum16(x):
    iota = lax.iota(jnp.int32, LANES)
    zero = jnp.zeros((LANES,), x.dtype)
    for k in (1, 2, 4, 8):
        g = _take(x, jnp.maximum(iota - k, 0))
        x = x + jnp.where(iota >= k, g, zero)
    return x


def _rsqrt(x):
    i = lax.bitcast_convert_type(x, jnp.int32)
    i = jnp.int32(0x5F3759DF) - lax.shift_right_arithmetic(i, 1)
    y = lax.bitcast_convert_type(i, jnp.float32)
    half = jnp.float32(0.5) * x
    for _ in range(2):
        y = y * (jnp.float32(1.5) - half * y * y)
    return y


def _sc_kernel(ids_hbm, word_hbm, pos_hbm, type_hbm, gamma_hbm, beta_hbm,
               out_hbm, ids_v, posid_v, pp_v, misc_v, w0, w1, w2,
               o0, o1, o2, sg0, sg1, sg2, so0, so1, so2):
    wid = lax.axis_index("s") * 2 + lax.axis_index("c")
    wbase = wid * TOK_W
    wbufs = (w0, w1, w2)
    obufs = (o0, o1, o2)
    sg = (sg0, sg1, sg2)
    so = (so0, so1, so2)

    pltpu.sync_copy(type_hbm, misc_v.at[pl.ds(0, 2)])
    pltpu.sync_copy(gamma_hbm, misc_v.at[2])
    pltpu.sync_copy(beta_hbm, misc_v.at[3])
    tv = [misc_v[1, pl.ds(v * LANES, LANES)] for v in range(HV)]
    gv = [misc_v[2, pl.ds(v * LANES, LANES)] for v in range(HV)]
    bv = [misc_v[3, pl.ds(v * LANES, LANES)] for v in range(HV)]

    pltpu.sync_copy(ids_hbm.at[pl.ds(wbase, TOK_W)],
                    ids_v.at[pl.ds(0, TOK_W)])

    pltpu.sync_copy(pos_hbm.at[pl.ds(0, PT)], pp_v)

    def pp_body(r, c):
        for v in range(HV):
            pp_v[r, pl.ds(v * LANES, LANES)] = (
                pp_v[r, pl.ds(v * LANES, LANES)] + tv[v])
        return c

    lax.fori_loop(0, PT, pp_body, jnp.int32(0))

    ones = jnp.ones((LANES,), jnp.int32)
    zeros = jnp.zeros((LANES,), jnp.int32)
    last = jnp.full((LANES,), LANES - 1, jnp.int32)

    def pos_body(r, c):
        run = zeros
        for v in range(NV):
            idv = ids_v[pl.ds(r * L + v * LANES, LANES)]
            m = jnp.where(idv != PAD, ones, zeros)
            cs = _cumsum16(m)
            posid_v[pl.ds(r * L + v * LANES, LANES)] = (
                (cs + run) * m + jnp.int32(PAD))
            run = run + _take(cs, last)
        return c

    lax.fori_loop(0, ROWS_PER_W, pos_body, jnp.int32(0))

    def gather_desc(c, b):
        return pltpu.make_async_copy(
            word_hbm.at[ids_v.at[pl.ds(c * CH, CH)]], wbufs[b], sg[b])

    def out_desc(c, b):
        return pltpu.make_async_copy(
            obufs[b], out_hbm.at[pl.ds(wbase + c * CH, CH)], so[b])

    gather_desc(0, 0).start()
    gather_desc(1, 1).start()

    def ring_body(g, c0):
        for b in range(3):
            c = 3 * g + b
            wb = wbufs[b]
            ob = obufs[b]

            @pl.when(c < NCH)
            def _():
                gather_desc(c, b).wait()

                def grp_body(gi, ci):
                    pvec = posid_v[pl.ds(c * CH + gi * LANES, LANES)]
                    base = gi * LANES
                    for j in range(LANES):
                        t = base + j
                        p = pvec[j]
                        s = None
                        sq = None
                        for v in range(HV):
                            x = (wb[t, pl.ds(v * LANES, LANES)]
                                 + pp_v[p, pl.ds(v * LANES, LANES)])
                            s = x if s is None else s + x
                            sq = x * x if sq is None else sq + x * x
                        tot = _allsum(s)
                        tot2 = _allsum(sq)
                        mean = tot * jnp.float32(1.0 / HID)
                        var = tot2 * jnp.float32(1.0 / HID) - mean * mean
                        inv = _rsqrt(var + jnp.float32(EPS))
                        for v in range(HV):
                            x = (wb[t, pl.ds(v * LANES, LANES)]
                                 + pp_v[p, pl.ds(v * LANES, LANES)])
                            ob[t, pl.ds(v * LANES, LANES)] = (
                                (x - mean) * inv * gv[v] + bv[v])
                    return ci

                lax.fori_loop(0, NG, grp_body, jnp.int32(0))
                out_desc(c, b).start()

            bn = (b + 2) % 3

            @pl.when((c + 2 < NCH) & (c >= 1))
            def _():
                out_desc(c - 1, bn).wait()

            @pl.when(c + 2 < NCH)
            def _():
                gather_desc(c + 2, bn).start()
        return c0

    lax.fori_loop(0, (NCH + 3) // 3, ring_body, jnp.int32(0))

    out_desc(NCH - 3, (NCH - 3) % 3).wait()
    out_desc(NCH - 2, (NCH - 2) % 3).wait()
    out_desc(NCH - 1, (NCH - 1) % 3).wait()


@functools.partial(jax.jit, static_argnames=())
def _impl(input_ids, word_embeddings, position_embeddings,
          token_type_embeddings, ln_gamma, ln_beta):
    mesh = plsc.VectorSubcoreMesh(core_axis_name="c", subcore_axis_name="s")
    f = pl.kernel(
        _sc_kernel,
        mesh=mesh,
        out_type=jax.ShapeDtypeStruct((B * L, HID), jnp.float32),
        scratch_types=[
            pltpu.VMEM((IDS_PAD,), jnp.int32),
            pltpu.VMEM((IDS_PAD,), jnp.int32),
            pltpu.VMEM((PT, HID), jnp.float32),
            pltpu.VMEM((4, HID), jnp.float32),
            pltpu.VMEM((CH, HID), jnp.float32),
            pltpu.VMEM((CH, HID), jnp.float32),
            pltpu.VMEM((CH, HID), jnp.float32),
            pltpu.VMEM((CH, HID), jnp.float32),
            pltpu.VMEM((CH, HID), jnp.float32),
            pltpu.VMEM((CH, HID), jnp.float32),
            pltpu.SemaphoreType.DMA,
            pltpu.SemaphoreType.DMA,
            pltpu.SemaphoreType.DMA,
            pltpu.SemaphoreType.DMA,
            pltpu.SemaphoreType.DMA,
            pltpu.SemaphoreType.DMA,
        ],
    )
    flat = f(input_ids.reshape(B * L), word_embeddings, position_embeddings,
             token_type_embeddings, ln_gamma, ln_beta)
    return flat.reshape(B, L, HID)


def kernel(input_ids, word_embeddings, position_embeddings,
           token_type_embeddings, ln_gamma, ln_beta):
    return _impl(input_ids.astype(jnp.int32), word_embeddings,
                 position_embeddings, token_type_embeddings,
                 ln_gamma, ln_beta)


# speedup vs baseline: 5.8135x; 5.8135x over previous
import functools

import jax
import jax.numpy as jnp
from jax import lax
from jax.experimental import pallas as pl
from jax.experimental.pallas import tpu as pltpu
from jax.experimental.pallas import tpu_sc as plsc

VOCAB = 100000
HID = 128
MAXPOS = 512
B = 1024
L = 200
PAD = 1
EPS = 1e-5

NW = 32
ROWS_PER_W = B // NW
TOK_W = ROWS_PER_W * L
LP = 208
NV = LP // 16
PT = 224
CH = 80
NCH = TOK_W // CH
NG = CH // 16
LANES = 16
HV = HID // LANES
IDS_PAD = TOK_W + LANES

_GATHER_DN = lax.GatherDimensionNumbers(
    offset_dims=(), collapsed_slice_dims=(0,), start_index_map=(0,))


def _take(x, idx):
    return lax.gather(x, idx[:, None], _GATHER_DN, slice_sizes=(1,),
                      mode=lax.GatherScatterMode.PROMISE_IN_BOUNDS)


def _allsum(x):
    iota = lax.iota(jnp.int32, LANES)
    for k in (1, 2, 4, 8):
        x = x + _take(x, iota ^ k)
    return x


def _cumsum16(x):
    iota = lax.iota(jnp.int32, LANES)
    zero = jnp.zeros((LANES,), x.dtype)
    for k in (1, 2, 4, 8):
        g = _take(x, jnp.maximum(iota - k, 0))
        x = x + jnp.where(iota >= k, g, zero)
    return x


def _rsqrt(x):
    i = lax.bitcast_convert_type(x, jnp.int32)
    i = jnp.int32(0x5F3759DF) - lax.shift_right_arithmetic(i, 1)
    y = lax.bitcast_convert_type(i, jnp.float32)
    half = jnp.float32(0.5) * x
    for _ in range(2):
        y = y * (jnp.float32(1.5) - half * y * y)
    return y


def _sc_kernel(ids_hbm, word_hbm, pos_hbm, type_hbm, gamma_hbm, beta_hbm,
               out_hbm, ids_v, posid_v, pp_v, misc_v, w0, w1, w2,
               o0, o1, o2, sg0, sg1, sg2, so0, so1, so2):
    wid = lax.axis_index("s") * 2 + lax.axis_index("c")
    wbase = wid * TOK_W
    wbufs = (w0, w1, w2)
    obufs = (o0, o1, o2)
    sg = (sg0, sg1, sg2)
    so = (so0, so1, so2)

    pltpu.sync_copy(type_hbm, misc_v.at[pl.ds(0, 2)])
    pltpu.sync_copy(gamma_hbm, misc_v.at[2])
    pltpu.sync_copy(beta_hbm, misc_v.at[3])
    tv = [misc_v[1, pl.ds(v * LANES, LANES)] for v in range(HV)]
    gv = [misc_v[2, pl.ds(v * LANES, LANES)] for v in range(HV)]
    bv = [misc_v[3, pl.ds(v * LANES, LANES)] for v in range(HV)]

    pltpu.sync_copy(ids_hbm.at[pl.ds(wbase, TOK_W)],
                    ids_v.at[pl.ds(0, TOK_W)])

    pltpu.sync_copy(pos_hbm.at[pl.ds(0, PT)], pp_v)

    def pp_body(r, c):
        for v in range(HV):
            pp_v[r, pl.ds(v * LANES, LANES)] = (
                pp_v[r, pl.ds(v * LANES, LANES)] + tv[v])
        return c

    lax.fori_loop(0, PT, pp_body, jnp.int32(0))

    ones = jnp.ones((LANES,), jnp.int32)
    zeros = jnp.zeros((LANES,), jnp.int32)
    last = jnp.full((LANES,), LANES - 1, jnp.int32)

    def pos_body(r, c):
        run = zeros
        for v in range(NV):
            idv = ids_v[pl.ds(r * L + v * LANES, LANES)]
            m = jnp.where(idv != PAD, ones, zeros)
            cs = _cumsum16(m)
            posid_v[pl.ds(r * L + v * LANES, LANES)] = (
                (cs + run) * m + jnp.int32(PAD))
            run = run + _take(cs, last)
        return c

    lax.fori_loop(0, ROWS_PER_W, pos_body, jnp.int32(0))

    def gather_desc(c, b):
        return pltpu.make_async_copy(
            word_hbm.at[ids_v.at[pl.ds(c * CH, CH)]], wbufs[b], sg[b])

    def out_desc(c, b):
        return pltpu.make_async_copy(
            obufs[b], out_hbm.at[pl.ds(wbase + c * CH, CH)], so[b])

    gather_desc(0, 0).start()
    gather_desc(1, 1).start()

    def ring_body(g, c0):
        for b in range(3):
            c = 3 * g + b
            wb = wbufs[b]
            ob = obufs[b]

            @pl.when(c < NCH)
            def _():
                gather_desc(c, b).wait()

                def grp_body(gi, ci):
                    pvec = posid_v[pl.ds(c * CH + gi * LANES, LANES)]
                    base = gi * LANES
                    for jj in range(LANES // 2):
                        ta = base + 2 * jj
                        tb = ta + 1
                        pa = pvec[2 * jj]
                        pb = pvec[2 * jj + 1]
                        xsa = []
                        xsb = []
                        sa = sb = sqa = sqb = None
                        for v in range(HV):
                            xa = (wb[ta, pl.ds(v * LANES, LANES)]
                                  + pp_v[pa, pl.ds(v * LANES, LANES)])
                            xb = (wb[tb, pl.ds(v * LANES, LANES)]
                                  + pp_v[pb, pl.ds(v * LANES, LANES)])
                            xsa.append(xa)
                            xsb.append(xb)
                            sa = xa if sa is None else sa + xa
                            sb = xb if sb is None else sb + xb
                            sqa = xa * xa if sqa is None else sqa + xa * xa
                            sqb = xb * xb if sqb is None else sqb + xb * xb
                        iota = lax.iota(jnp.int32, LANES)
                        for k in (1, 2, 4, 8):
                            sa = sa + _take(sa, iota ^ k)
                            sb = sb + _take(sb, iota ^ k)
                            sqa = sqa + _take(sqa, iota ^ k)
                            sqb = sqb + _take(sqb, iota ^ k)
                        meana = sa * jnp.float32(1.0 / HID)
                        meanb = sb * jnp.float32(1.0 / HID)
                        vara = (sqa * jnp.float32(1.0 / HID)
                                - meana * meana + jnp.float32(EPS))
                        varb = (sqb * jnp.float32(1.0 / HID)
                                - meanb * meanb + jnp.float32(EPS))
                        ia = lax.bitcast_convert_type(vara, jnp.int32)
                        ib = lax.bitcast_convert_type(varb, jnp.int32)
                        ia = (jnp.int32(0x5F3759DF)
                              - lax.shift_right_arithmetic(ia, 1))
                        ib = (jnp.int32(0x5F3759DF)
                              - lax.shift_right_arithmetic(ib, 1))
                        ya = lax.bitcast_convert_type(ia, jnp.float32)
                        yb = lax.bitcast_convert_type(ib, jnp.float32)
                        ha = jnp.float32(0.5) * vara
                        hb = jnp.float32(0.5) * varb
                        for _ in range(2):
                            ya = ya * (jnp.float32(1.5) - ha * ya * ya)
                            yb = yb * (jnp.float32(1.5) - hb * yb * yb)
                        for v in range(HV):
                            ob[ta, pl.ds(v * LANES, LANES)] = (
                                (xsa[v] - meana) * ya * gv[v] + bv[v])
                            ob[tb, pl.ds(v * LANES, LANES)] = (
                                (xsb[v] - meanb) * yb * gv[v] + bv[v])
                    return ci

                lax.fori_loop(0, NG, grp_body, jnp.int32(0))
                out_desc(c, b).start()

            bn = (b + 2) % 3

            @pl.when((c + 2 < NCH) & (c >= 1))
            def _():
                out_desc(c - 1, bn).wait()

            @pl.when(c + 2 < NCH)
            def _():
                gather_desc(c + 2, bn).start()
        return c0

    lax.fori_loop(0, (NCH + 3) // 3, ring_body, jnp.int32(0))

    out_desc(NCH - 3, (NCH - 3) % 3).wait()
    out_desc(NCH - 2, (NCH - 2) % 3).wait()
    out_desc(NCH - 1, (NCH - 1) % 3).wait()


@functools.partial(jax.jit, static_argnames=())
def _impl(input_ids, word_embeddings, position_embeddings,
          token_type_embeddings, ln_gamma, ln_beta):
    mesh = plsc.VectorSubcoreMesh(core_axis_name="c", subcore_axis_name="s")
    f = pl.kernel(
        _sc_kernel,
        mesh=mesh,
        out_type=jax.ShapeDtypeStruct((B * L, HID), jnp.float32),
        scratch_types=[
            pltpu.VMEM((IDS_PAD,), jnp.int32),
            pltpu.VMEM((IDS_PAD,), jnp.int32),
            pltpu.VMEM((PT, HID), jnp.float32),
            pltpu.VMEM((4, HID), jnp.float32),
            pltpu.VMEM((CH, HID), jnp.float32),
            pltpu.VMEM((CH, HID), jnp.float32),
            pltpu.VMEM((CH, HID), jnp.float32),
            pltpu.VMEM((CH, HID), jnp.float32),
            pltpu.VMEM((CH, HID), jnp.float32),
            pltpu.VMEM((CH, HID), jnp.float32),
            pltpu.SemaphoreType.DMA,
            pltpu.SemaphoreType.DMA,
            pltpu.SemaphoreType.DMA,
            pltpu.SemaphoreType.DMA,
            pltpu.SemaphoreType.DMA,
            pltpu.SemaphoreType.DMA,
        ],
    )
    flat = f(input_ids.reshape(B * L), word_embeddings, position_embeddings,
             token_type_embeddings, ln_gamma, ln_beta)
    return flat.reshape(B, L, HID)


def kernel(input_ids, word_embeddings, position_embeddings,
           token_type_embeddings, ln_gamma, ln_beta):
    return _impl(input_ids.astype(jnp.int32), word_embeddings,
                 position_embeddings, token_type_embeddings,
                 ln_gamma, ln_beta)


# speedup vs baseline: 6.9256x; 1.1913x over previous
import functools

import jax
import jax.numpy as jnp
from jax import lax
from jax.experimental import pallas as pl
from jax.experimental.pallas import tpu as pltpu
from jax.experimental.pallas import tpu_sc as plsc

VOCAB = 100000
HID = 128
MAXPOS = 512
B = 1024
L = 200
PAD = 1
EPS = 1e-5

NW = 32
ROWS_PER_W = B // NW
TOK_W = ROWS_PER_W * L
LP = 208
NV = LP // 16
PT = 224
CH = 80
NCH = TOK_W // CH
NG = CH // 16
LANES = 16
HV = HID // LANES
IDS_PAD = TOK_W + LANES

_GATHER_DN = lax.GatherDimensionNumbers(
    offset_dims=(), collapsed_slice_dims=(0,), start_index_map=(0,))


def _take(x, idx):
    return lax.gather(x, idx[:, None], _GATHER_DN, slice_sizes=(1,),
                      mode=lax.GatherScatterMode.PROMISE_IN_BOUNDS)


def _allsum(x):
    iota = lax.iota(jnp.int32, LANES)
    for k in (1, 2, 4, 8):
        x = x + _take(x, iota ^ k)
    return x


def _cumsum16(x):
    iota = lax.iota(jnp.int32, LANES)
    zero = jnp.zeros((LANES,), x.dtype)
    for k in (1, 2, 4, 8):
        g = _take(x, jnp.maximum(iota - k, 0))
        x = x + jnp.where(iota >= k, g, zero)
    return x


def _rsqrt(x):
    i = lax.bitcast_convert_type(x, jnp.int32)
    i = jnp.int32(0x5F3759DF) - lax.shift_right_arithmetic(i, 1)
    y = lax.bitcast_convert_type(i, jnp.float32)
    half = jnp.float32(0.5) * x
    for _ in range(2):
        y = y * (jnp.float32(1.5) - half * y * y)
    return y


def _sc_kernel(ids_hbm, word_hbm, pos_hbm, type_hbm, gamma_hbm, beta_hbm,
               out_hbm, ids_v, posid_v, pp_v, misc_v, w0, w1, w2,
               o0, o1, o2, sg0, sg1, sg2, so0, so1, so2):
    wid = lax.axis_index("s") * 2 + lax.axis_index("c")
    wbase = wid * TOK_W
    wbufs = (w0, w1, w2)
    obufs = (o0, o1, o2)
    sg = (sg0, sg1, sg2)
    so = (so0, so1, so2)

    pltpu.sync_copy(type_hbm, misc_v.at[pl.ds(0, 2)])
    pltpu.sync_copy(gamma_hbm, misc_v.at[2])
    pltpu.sync_copy(beta_hbm, misc_v.at[3])
    tv = [misc_v[1, pl.ds(v * LANES, LANES)] for v in range(HV)]
    gv = [misc_v[2, pl.ds(v * LANES, LANES)] for v in range(HV)]
    bv = [misc_v[3, pl.ds(v * LANES, LANES)] for v in range(HV)]

    pltpu.sync_copy(ids_hbm.at[pl.ds(wbase, TOK_W)],
                    ids_v.at[pl.ds(0, TOK_W)])

    pltpu.sync_copy(pos_hbm.at[pl.ds(0, PT)], pp_v)

    def pp_body(r, c):
        for v in range(HV):
            pp_v[r, pl.ds(v * LANES, LANES)] = (
                pp_v[r, pl.ds(v * LANES, LANES)] + tv[v])
        return c

    lax.fori_loop(0, PT, pp_body, jnp.int32(0))

    ones = jnp.ones((LANES,), jnp.int32)
    zeros = jnp.zeros((LANES,), jnp.int32)
    last = jnp.full((LANES,), LANES - 1, jnp.int32)

    def pos_body(r, c):
        run = zeros
        for v in range(NV):
            idv = ids_v[pl.ds(r * L + v * LANES, LANES)]
            m = jnp.where(idv != PAD, ones, zeros)
            cs = _cumsum16(m)
            posid_v[pl.ds(r * L + v * LANES, LANES)] = (
                (cs + run) * m + jnp.int32(PAD))
            run = run + _take(cs, last)
        return c

    lax.fori_loop(0, ROWS_PER_W, pos_body, jnp.int32(0))

    def gather_desc(c, b):
        return pltpu.make_async_copy(
            word_hbm.at[ids_v.at[pl.ds(c * CH, CH)]], wbufs[b], sg[b])

    def out_desc(c, b):
        return pltpu.make_async_copy(
            obufs[b], out_hbm.at[pl.ds(wbase + c * CH, CH)], so[b])

    gather_desc(0, 0).start()
    gather_desc(1, 1).start()

    def ring_body(g, c0):
        for b in range(3):
            c = 3 * g + b
            wb = wbufs[b]
            ob = obufs[b]

            @pl.when(c < NCH)
            def _():
                gather_desc(c, b).wait()

                def grp_body(gi, ci):
                    pvec = posid_v[pl.ds(c * CH + gi * LANES, LANES)]
                    base = gi * LANES
                    IL = 4
                    iota = lax.iota(jnp.int32, LANES)
                    for jj in range(LANES // IL):
                        ts = [base + IL * jj + u for u in range(IL)]
                        ps = [pvec[IL * jj + u] for u in range(IL)]
                        xss = [[] for _ in range(IL)]
                        ss = [None] * IL
                        sqs = [None] * IL
                        for v in range(HV):
                            for u in range(IL):
                                x = (wb[ts[u], pl.ds(v * LANES, LANES)]
                                     + pp_v[ps[u], pl.ds(v * LANES, LANES)])
                                xss[u].append(x)
                                ss[u] = x if ss[u] is None else ss[u] + x
                                sqs[u] = (x * x if sqs[u] is None
                                          else sqs[u] + x * x)
                        for k in (1, 2, 4, 8):
                            for u in range(IL):
                                ss[u] = ss[u] + _take(ss[u], iota ^ k)
                                sqs[u] = sqs[u] + _take(sqs[u], iota ^ k)
                        means = [ss[u] * jnp.float32(1.0 / HID)
                                 for u in range(IL)]
                        vrs = [(sqs[u] * jnp.float32(1.0 / HID)
                                - means[u] * means[u] + jnp.float32(EPS))
                               for u in range(IL)]
                        ys = []
                        hs = []
                        for u in range(IL):
                            iv = lax.bitcast_convert_type(vrs[u], jnp.int32)
                            iv = (jnp.int32(0x5F3759DF)
                                  - lax.shift_right_arithmetic(iv, 1))
                            ys.append(lax.bitcast_convert_type(
                                iv, jnp.float32))
                            hs.append(jnp.float32(0.5) * vrs[u])
                        for _ in range(2):
                            for u in range(IL):
                                ys[u] = ys[u] * (jnp.float32(1.5)
                                                 - hs[u] * ys[u] * ys[u])
                        for v in range(HV):
                            for u in range(IL):
                                ob[ts[u], pl.ds(v * LANES, LANES)] = (
                                    (xss[u][v] - means[u]) * ys[u] * gv[v]
                                    + bv[v])
                    return ci

                lax.fori_loop(0, NG, grp_body, jnp.int32(0))
                out_desc(c, b).start()

            bn = (b + 2) % 3

            @pl.when((c + 2 < NCH) & (c >= 1))
            def _():
                out_desc(c - 1, bn).wait()

            @pl.when(c + 2 < NCH)
            def _():
                gather_desc(c + 2, bn).start()
        return c0

    lax.fori_loop(0, (NCH + 3) // 3, ring_body, jnp.int32(0))

    out_desc(NCH - 3, (NCH - 3) % 3).wait()
    out_desc(NCH - 2, (NCH - 2) % 3).wait()
    out_desc(NCH - 1, (NCH - 1) % 3).wait()


@functools.partial(jax.jit, static_argnames=())
def _impl(input_ids, word_embeddings, position_embeddings,
          token_type_embeddings, ln_gamma, ln_beta):
    mesh = plsc.VectorSubcoreMesh(core_axis_name="c", subcore_axis_name="s")
    f = pl.kernel(
        _sc_kernel,
        mesh=mesh,
        out_type=jax.ShapeDtypeStruct((B * L, HID), jnp.float32),
        scratch_types=[
            pltpu.VMEM((IDS_PAD,), jnp.int32),
            pltpu.VMEM((IDS_PAD,), jnp.int32),
            pltpu.VMEM((PT, HID), jnp.float32),
            pltpu.VMEM((4, HID), jnp.float32),
            pltpu.VMEM((CH, HID), jnp.float32),
            pltpu.VMEM((CH, HID), jnp.float32),
            pltpu.VMEM((CH, HID), jnp.float32),
            pltpu.VMEM((CH, HID), jnp.float32),
            pltpu.VMEM((CH, HID), jnp.float32),
            pltpu.VMEM((CH, HID), jnp.float32),
            pltpu.SemaphoreType.DMA,
            pltpu.SemaphoreType.DMA,
            pltpu.SemaphoreType.DMA,
            pltpu.SemaphoreType.DMA,
            pltpu.SemaphoreType.DMA,
            pltpu.SemaphoreType.DMA,
        ],
    )
    flat = f(input_ids.reshape(B * L), word_embeddings, position_embeddings,
             token_type_embeddings, ln_gamma, ln_beta)
    return flat.reshape(B, L, HID)


def kernel(input_ids, word_embeddings, position_embeddings,
           token_type_embeddings, ln_gamma, ln_beta):
    return _impl(input_ids.astype(jnp.int32), word_embeddings,
                 position_embeddings, token_type_embeddings,
                 ln_gamma, ln_beta)


# speedup vs baseline: 7.3112x; 1.0557x over previous
import functools

import jax
import jax.numpy as jnp
from jax import lax
from jax.experimental import pallas as pl
from jax.experimental.pallas import tpu as pltpu
from jax.experimental.pallas import tpu_sc as plsc

VOCAB = 100000
HID = 128
MAXPOS = 512
B = 1024
L = 200
PAD = 1
EPS = 1e-5

NW = 32
ROWS_PER_W = B // NW
TOK_W = ROWS_PER_W * L
LP = 208
NV = LP // 16
PT = 224
CH = 80
NCH = TOK_W // CH
NG = CH // 16
LANES = 16
HV = HID // LANES
IDS_PAD = TOK_W + LANES

_GATHER_DN = lax.GatherDimensionNumbers(
    offset_dims=(), collapsed_slice_dims=(0,), start_index_map=(0,))


def _take(x, idx):
    return lax.gather(x, idx[:, None], _GATHER_DN, slice_sizes=(1,),
                      mode=lax.GatherScatterMode.PROMISE_IN_BOUNDS)


def _allsum(x):
    iota = lax.iota(jnp.int32, LANES)
    for k in (1, 2, 4, 8):
        x = x + _take(x, iota ^ k)
    return x


def _cumsum16(x):
    iota = lax.iota(jnp.int32, LANES)
    zero = jnp.zeros((LANES,), x.dtype)
    for k in (1, 2, 4, 8):
        g = _take(x, jnp.maximum(iota - k, 0))
        x = x + jnp.where(iota >= k, g, zero)
    return x


def _rsqrt(x):
    i = lax.bitcast_convert_type(x, jnp.int32)
    i = jnp.int32(0x5F3759DF) - lax.shift_right_arithmetic(i, 1)
    y = lax.bitcast_convert_type(i, jnp.float32)
    half = jnp.float32(0.5) * x
    for _ in range(1):
        y = y * (jnp.float32(1.5) - half * y * y)
    return y


def _sc_kernel(ids_hbm, word_hbm, pos_hbm, type_hbm, gamma_hbm, beta_hbm,
               out_hbm, ids_v, posid_v, pp_v, misc_v, w0, w1, w2,
               o0, o1, o2, sg0, sg1, sg2, so0, so1, so2):
    wid = lax.axis_index("s") * 2 + lax.axis_index("c")
    wbase = wid * TOK_W
    wbufs = (w0, w1, w2)
    obufs = (o0, o1, o2)
    sg = (sg0, sg1, sg2)
    so = (so0, so1, so2)

    pltpu.sync_copy(type_hbm, misc_v.at[pl.ds(0, 2)])
    pltpu.sync_copy(gamma_hbm, misc_v.at[2])
    pltpu.sync_copy(beta_hbm, misc_v.at[3])
    tv = [misc_v[1, pl.ds(v * LANES, LANES)] for v in range(HV)]
    gv = [misc_v[2, pl.ds(v * LANES, LANES)] for v in range(HV)]
    bv = [misc_v[3, pl.ds(v * LANES, LANES)] for v in range(HV)]

    pltpu.sync_copy(ids_hbm.at[pl.ds(wbase, TOK_W)],
                    ids_v.at[pl.ds(0, TOK_W)])

    pltpu.sync_copy(pos_hbm.at[pl.ds(0, PT)], pp_v)

    def pp_body(r, c):
        for v in range(HV):
            pp_v[r, pl.ds(v * LANES, LANES)] = (
                pp_v[r, pl.ds(v * LANES, LANES)] + tv[v])
        return c

    lax.fori_loop(0, PT, pp_body, jnp.int32(0))

    ones = jnp.ones((LANES,), jnp.int32)
    zeros = jnp.zeros((LANES,), jnp.int32)
    last = jnp.full((LANES,), LANES - 1, jnp.int32)

    def pos_body(r, c):
        run = zeros
        for v in range(NV):
            idv = ids_v[pl.ds(r * L + v * LANES, LANES)]
            m = jnp.where(idv != PAD, ones, zeros)
            cs = _cumsum16(m)
            posid_v[pl.ds(r * L + v * LANES, LANES)] = (
                (cs + run) * m + jnp.int32(PAD))
            run = run + _take(cs, last)
        return c

    lax.fori_loop(0, ROWS_PER_W, pos_body, jnp.int32(0))

    def gather_desc(c, b):
        return pltpu.make_async_copy(
            word_hbm.at[ids_v.at[pl.ds(c * CH, CH)]], wbufs[b], sg[b])

    def out_desc(c, b):
        return pltpu.make_async_copy(
            obufs[b], out_hbm.at[pl.ds(wbase + c * CH, CH)], so[b])

    gather_desc(0, 0).start()
    gather_desc(1, 1).start()

    def ring_body(g, c0):
        for b in range(3):
            c = 3 * g + b
            wb = wbufs[b]
            ob = obufs[b]

            @pl.when(c < NCH)
            def _():
                gather_desc(c, b).wait()

                def grp_body(gi, ci):
                    pvec = posid_v[pl.ds(c * CH + gi * LANES, LANES)]
                    base = gi * LANES
                    IL = 4
                    iota = lax.iota(jnp.int32, LANES)
                    for jj in range(LANES // IL):
                        ts = [base + IL * jj + u for u in range(IL)]
                        ps = [pvec[IL * jj + u] for u in range(IL)]
                        xss = [[] for _ in range(IL)]
                        ss = [None] * IL
                        sqs = [None] * IL
                        for v in range(HV):
                            for u in range(IL):
                                x = (wb[ts[u], pl.ds(v * LANES, LANES)]
                                     + pp_v[ps[u], pl.ds(v * LANES, LANES)])
                                xss[u].append(x)
                                ss[u] = x if ss[u] is None else ss[u] + x
                                sqs[u] = (x * x if sqs[u] is None
                                          else sqs[u] + x * x)
                        for k in (1, 2, 4, 8):
                            for u in range(IL):
                                ss[u] = ss[u] + _take(ss[u], iota ^ k)
                                sqs[u] = sqs[u] + _take(sqs[u], iota ^ k)
                        means = [ss[u] * jnp.float32(1.0 / HID)
                                 for u in range(IL)]
                        vrs = [(sqs[u] * jnp.float32(1.0 / HID)
                                - means[u] * means[u] + jnp.float32(EPS))
                               for u in range(IL)]
                        ys = []
                        hs = []
                        for u in range(IL):
                            iv = lax.bitcast_convert_type(vrs[u], jnp.int32)
                            iv = (jnp.int32(0x5F3759DF)
                                  - lax.shift_right_arithmetic(iv, 1))
                            ys.append(lax.bitcast_convert_type(
                                iv, jnp.float32))
                            hs.append(jnp.float32(0.5) * vrs[u])
                        for _ in range(1):
                            for u in range(IL):
                                ys[u] = ys[u] * (jnp.float32(1.5)
                                                 - hs[u] * ys[u] * ys[u])
                        for v in range(HV):
                            for u in range(IL):
                                ob[ts[u], pl.ds(v * LANES, LANES)] = (
                                    (xss[u][v] - means[u]) * ys[u] * gv[v]
                                    + bv[v])
                    return ci

                lax.fori_loop(0, NG, grp_body, jnp.int32(0))
                out_desc(c, b).start()

            bn = (b + 2) % 3

            @pl.when((c + 2 < NCH) & (c >= 1))
            def _():
                out_desc(c - 1, bn).wait()

            @pl.when(c + 2 < NCH)
            def _():
                gather_desc(c + 2, bn).start()
        return c0

    lax.fori_loop(0, (NCH + 3) // 3, ring_body, jnp.int32(0))

    out_desc(NCH - 3, (NCH - 3) % 3).wait()
    out_desc(NCH - 2, (NCH - 2) % 3).wait()
    out_desc(NCH - 1, (NCH - 1) % 3).wait()


@functools.partial(jax.jit, static_argnames=())
def _impl(input_ids, word_embeddings, position_embeddings,
          token_type_embeddings, ln_gamma, ln_beta):
    mesh = plsc.VectorSubcoreMesh(core_axis_name="c", subcore_axis_name="s")
    f = pl.kernel(
        _sc_kernel,
        mesh=mesh,
        out_type=jax.ShapeDtypeStruct((B * L, HID), jnp.float32),
        scratch_types=[
            pltpu.VMEM((IDS_PAD,), jnp.int32),
            pltpu.VMEM((IDS_PAD,), jnp.int32),
            pltpu.VMEM((PT, HID), jnp.float32),
            pltpu.VMEM((4, HID), jnp.float32),
            pltpu.VMEM((CH, HID), jnp.float32),
            pltpu.VMEM((CH, HID), jnp.float32),
            pltpu.VMEM((CH, HID), jnp.float32),
            pltpu.VMEM((CH, HID), jnp.float32),
            pltpu.VMEM((CH, HID), jnp.float32),
            pltpu.VMEM((CH, HID), jnp.float32),
            pltpu.SemaphoreType.DMA,
            pltpu.SemaphoreType.DMA,
            pltpu.SemaphoreType.DMA,
            pltpu.SemaphoreType.DMA,
            pltpu.SemaphoreType.DMA,
            pltpu.SemaphoreType.DMA,
        ],
    )
    flat = f(input_ids.reshape(B * L), word_embeddings, position_embeddings,
             token_type_embeddings, ln_gamma, ln_beta)
    return flat.reshape(B, L, HID)


def kernel(input_ids, word_embeddings, position_embeddings,
           token_type_embeddings, ln_gamma, ln_beta):
    return _impl(input_ids.astype(jnp.int32), word_embeddings,
                 position_embeddings, token_type_embeddings,
                 ln_gamma, ln_beta)
